# baseline (device time: 46486 ns/iter reference)
import jax
import jax.numpy as jnp
from jax import lax
from jax.experimental import pallas as pl
from jax.experimental.pallas import tpu as pltpu

N_DEV = 4


def kernel(x, Wg, Wu, Wd):
    m, d_in = x.shape
    d_out = Wd.shape[1]
    ch = m // N_DEV

    def body(x_ref, wg_ref, wu_ref, wd_ref, out_ref,
             send_buf, rs_buf, own_buf,
             rs_send_sems, rs_recv_sems, ag_send_sems, ag_recv_sems):
        d = lax.axis_index("i")

        barrier_sem = pltpu.get_barrier_semaphore()
        for j in range(1, N_DEV):
            pl.semaphore_signal(
                barrier_sem, inc=1,
                device_id=((d + j) % N_DEV,),
                device_id_type=pl.DeviceIdType.MESH,
            )
        pl.semaphore_wait(barrier_sem, N_DEV - 1)

        wg = wg_ref[...]
        wu = wu_ref[...]
        wd = wd_ref[...]

        def partial_chunk(c):
            xc = x_ref[pl.ds(c * ch, ch), :]
            gate = jnp.dot(xc, wg, preferred_element_type=jnp.float32)
            up = jnp.dot(xc, wu, preferred_element_type=jnp.float32)
            h = gate * (up * jax.nn.sigmoid(up))
            return jnp.dot(h, wd, preferred_element_type=jnp.float32)

        rs = []
        for j in range(1, N_DEV):
            c = (d + j) % N_DEV
            send_buf[j - 1, :, :] = partial_chunk(c)
            rdma = pltpu.make_async_remote_copy(
                src_ref=send_buf.at[j - 1],
                dst_ref=rs_buf.at[N_DEV - 1 - j],
                send_sem=rs_send_sems.at[j - 1],
                recv_sem=rs_recv_sems.at[N_DEV - 1 - j],
                device_id=(c,),
                device_id_type=pl.DeviceIdType.MESH,
            )
            rdma.start()
            rs.append(rdma)

        acc = partial_chunk(d)
        for rdma in rs:
            rdma.wait_recv()
        acc = acc + rs_buf[0] + rs_buf[1] + rs_buf[2]
        own_buf[...] = acc
        out_ref[pl.ds(d * ch, ch), :] = acc

        ag = []
        for j in range(1, N_DEV):
            rdma = pltpu.make_async_remote_copy(
                src_ref=own_buf,
                dst_ref=out_ref.at[pl.ds(d * ch, ch)],
                send_sem=ag_send_sems.at[j - 1],
                recv_sem=ag_recv_sems.at[N_DEV - 1 - j],
                device_id=((d + j) % N_DEV,),
                device_id_type=pl.DeviceIdType.MESH,
            )
            rdma.start()
            ag.append(rdma)
        for rdma in ag:
            rdma.wait_recv()
        for rdma in rs:
            rdma.wait_send()
        for rdma in ag:
            rdma.wait_send()

    return pl.pallas_call(
        body,
        out_shape=jax.ShapeDtypeStruct((m, d_out), jnp.float32),
        in_specs=[pl.BlockSpec(memory_space=pltpu.VMEM)] * 4,
        out_specs=pl.BlockSpec(memory_space=pltpu.VMEM),
        scratch_shapes=[
            pltpu.VMEM((N_DEV - 1, ch, d_out), jnp.float32),
            pltpu.VMEM((N_DEV - 1, ch, d_out), jnp.float32),
            pltpu.VMEM((ch, d_out), jnp.float32),
            pltpu.SemaphoreType.DMA((N_DEV - 1,)),
            pltpu.SemaphoreType.DMA((N_DEV - 1,)),
            pltpu.SemaphoreType.DMA((N_DEV - 1,)),
            pltpu.SemaphoreType.DMA((N_DEV - 1,)),
        ],
        compiler_params=pltpu.CompilerParams(collective_id=0),
    )(x, Wg, Wu, Wd)


# device time: 19689 ns/iter; 2.3610x vs baseline; 2.3610x over previous
import jax
import jax.numpy as jnp
from jax import lax
from jax.experimental import pallas as pl
from jax.experimental.pallas import tpu as pltpu

N_DEV = 4


def kernel(x, Wg, Wu, Wd):
    m, d_in = x.shape
    d_out = Wd.shape[1]
    ch = m // N_DEV
    sub = ch // 2

    def body(x_ref, wg_ref, wu_ref, wd_ref, out_ref,
             send_buf, rs_buf, own_buf,
             rs_send_sems, rs_recv_sems, ag_send_sems, ag_recv_sems):
        d = lax.axis_index("i")

        barrier_sem = pltpu.get_barrier_semaphore()
        for j in range(1, N_DEV):
            pl.semaphore_signal(
                barrier_sem, inc=1,
                device_id=((d + j) % N_DEV,),
                device_id_type=pl.DeviceIdType.MESH,
            )
        pl.semaphore_wait(barrier_sem, N_DEV - 1)

        wg = wg_ref[...]
        wu = wu_ref[...]
        wd = wd_ref[...]

        def partial_rows(c, t):
            xc = x_ref[pl.ds(c * ch + t * sub, sub), :]
            gate = jnp.dot(xc, wg, preferred_element_type=jnp.float32)
            up = jnp.dot(xc, wu, preferred_element_type=jnp.float32)
            h = gate * (up * jax.nn.sigmoid(up))
            return jnp.dot(h, wd, preferred_element_type=jnp.float32)

        def rs_send(j, t):
            c = (d + j) % N_DEV
            slot = t * 3 + (j - 1)
            send_buf[slot, :, :] = partial_rows(c, t)
            rdma = pltpu.make_async_remote_copy(
                src_ref=send_buf.at[slot],
                dst_ref=rs_buf.at[t * 3 + (N_DEV - 1 - j)],
                send_sem=rs_send_sems.at[slot],
                recv_sem=rs_recv_sems.at[t * 3 + (N_DEV - 1 - j)],
                device_id=(c,),
                device_id_type=pl.DeviceIdType.MESH,
            )
            rdma.start()
            return rdma

        def ag_send(j, t):
            rdma = pltpu.make_async_remote_copy(
                src_ref=own_buf.at[t],
                dst_ref=out_ref.at[pl.ds(d * ch + t * sub, sub)],
                send_sem=ag_send_sems.at[t * 3 + (j - 1)],
                recv_sem=ag_recv_sems.at[t * 3 + (N_DEV - 1 - j)],
                device_id=((d + j) % N_DEV,),
                device_id_type=pl.DeviceIdType.MESH,
            )
            rdma.start()
            return rdma

        rs_a = [rs_send(j, 0) for j in range(1, N_DEV)]
        acc_a = partial_rows(d, 0)

        rs_b = [rs_send(1, 1)]

        for rdma in rs_a:
            rdma.wait_recv()
        acc_a = acc_a + rs_buf[0] + rs_buf[1] + rs_buf[2]
        own_buf[0, :, :] = acc_a
        out_ref[pl.ds(d * ch, sub), :] = acc_a
        ag_a = [ag_send(j, 0) for j in range(1, N_DEV)]

        rs_b += [rs_send(j, 1) for j in range(2, N_DEV)]
        acc_b = partial_rows(d, 1)
        for rdma in rs_b:
            rdma.wait_recv()
        acc_b = acc_b + rs_buf[3] + rs_buf[4] + rs_buf[5]
        own_buf[1, :, :] = acc_b
        out_ref[pl.ds(d * ch + sub, sub), :] = acc_b
        ag_b = [ag_send(j, 1) for j in range(1, N_DEV)]

        for rdma in ag_a + ag_b:
            rdma.wait_recv()
        for rdma in rs_a + rs_b + ag_a + ag_b:
            rdma.wait_send()

    return pl.pallas_call(
        body,
        out_shape=jax.ShapeDtypeStruct((m, d_out), jnp.float32),
        in_specs=[pl.BlockSpec(memory_space=pltpu.VMEM)] * 4,
        out_specs=pl.BlockSpec(memory_space=pltpu.VMEM),
        scratch_shapes=[
            pltpu.VMEM((6, sub, d_out), jnp.float32),
            pltpu.VMEM((6, sub, d_out), jnp.float32),
            pltpu.VMEM((2, sub, d_out), jnp.float32),
            pltpu.SemaphoreType.DMA((6,)),
            pltpu.SemaphoreType.DMA((6,)),
            pltpu.SemaphoreType.DMA((6,)),
            pltpu.SemaphoreType.DMA((6,)),
        ],
        compiler_params=pltpu.CompilerParams(collective_id=0),
    )(x, Wg, Wu, Wd)
